# Initial kernel scaffold; baseline (speedup 1.0000x reference)
import jax
import jax.numpy as jnp
from jax.experimental import pallas as pl


def kernel(x, edge_index, W1, b1, Wl, bl, Wr, W2, b2):
    raise NotImplementedError("write your pallas kernel here")


# speedup vs baseline: 35.6305x; 35.6305x over previous
import functools

import jax
import jax.numpy as jnp
from jax import lax
from jax.experimental import pallas as pl
from jax.experimental.pallas import tpu as pltpu
from jax.experimental.pallas import tpu_sc as plsc

_NTILES = 16
_NCORES = 2
_NW = _NTILES * _NCORES


def _round_up(a, b):
    return (a + b - 1) // b * b


@functools.lru_cache(maxsize=None)
def _build_sc_agg(n_nodes, n_edges):
    n_acc = _round_up(n_nodes + 96, _NTILES * 128)
    zrows = n_acc // _NTILES

    ept = n_edges // _NW
    EB = 4000
    NBLK = ept // EB
    GCH = 400
    CCH = 800
    G = 2

    mesh = plsc.VectorSubcoreMesh(core_axis_name="c", subcore_axis_name="s")

    @functools.partial(
        pl.kernel,
        out_type=[
            jax.ShapeDtypeStruct((_NCORES, n_acc, 16), jnp.float32),
            jax.ShapeDtypeStruct((_NCORES * n_acc,), jnp.float32),
        ],
        mesh=mesh,
        compiler_params=pltpu.CompilerParams(use_tc_tiling_on_sc=False),
        scratch_types=[
            pltpu.VMEM((EB,), jnp.int32),
            pltpu.VMEM((EB,), jnp.int32),
            pltpu.VMEM((G, GCH, 16), jnp.float32),
            pltpu.VMEM((CCH,), jnp.float32),
            pltpu.VMEM_SHARED((n_acc, 16), jnp.float32),
            pltpu.VMEM_SHARED((n_acc,), jnp.float32),
            pltpu.SemaphoreType.DMA((G,)),
        ],
    )
    def sc_agg(h_hbm, src_hbm, dst_hbm, zsum_hbm, zcnt_hbm, ones_hbm,
               out_sum, out_cnt,
               src_v, dst_v, rows, ones_v, sum_sh, cnt_sh, gsem):
        c = lax.axis_index("c")
        s = lax.axis_index("s")

        pltpu.sync_copy(zsum_hbm, sum_sh.at[pl.ds(s * zrows, zrows)])
        pltpu.sync_copy(zcnt_hbm, cnt_sh.at[pl.ds(s * zrows, zrows)])
        pltpu.sync_copy(ones_hbm, ones_v)
        plsc.subcore_barrier()

        base = (c * _NTILES + s) * ept
        nch = EB // GCH
        for blk in range(NBLK):
            off = base + blk * EB
            pltpu.sync_copy(src_hbm.at[pl.ds(off, EB)], src_v)
            pltpu.sync_copy(dst_hbm.at[pl.ds(off, EB)], dst_v)

            for g in range(G):
                pltpu.async_copy(h_hbm.at[src_v.at[pl.ds(g * GCH, GCH)]],
                                 rows.at[g], gsem.at[g])

            def gstep(j, carry):
                slot = lax.rem(j, G)
                pltpu.make_async_copy(
                    h_hbm.at[src_v.at[pl.ds(j * GCH, GCH)]],
                    rows.at[slot], gsem.at[slot]).wait()
                pltpu.sync_copy(rows.at[slot],
                                sum_sh.at[dst_v.at[pl.ds(j * GCH, GCH)]],
                                add=True)

                @pl.when(j + G < nch)
                def _fire():
                    pltpu.async_copy(
                        h_hbm.at[src_v.at[pl.ds((j + G) * GCH, GCH)]],
                        rows.at[slot], gsem.at[slot])
                return carry

            lax.fori_loop(0, nch, gstep, 0)

            def cstep(i, carry):
                pltpu.sync_copy(ones_v,
                                cnt_sh.at[dst_v.at[pl.ds(i * CCH, CCH)]],
                                add=True)
                return carry

            lax.fori_loop(0, EB // CCH, cstep, 0)

        plsc.subcore_barrier()
        pltpu.sync_copy(sum_sh.at[pl.ds(s * zrows, zrows)],
                        out_sum.at[c, pl.ds(s * zrows, zrows)])
        pltpu.sync_copy(cnt_sh.at[pl.ds(s * zrows, zrows)],
                        out_cnt.at[pl.ds(c * n_acc + s * zrows, zrows)])

    return sc_agg, n_acc


def _lin1(x, w1t, b1):
    n = x.shape[0]
    r = 2048
    grid = pl.cdiv(n, r)
    return pl.pallas_call(
        lambda x_ref, w_ref, b_ref, o_ref: o_ref.__setitem__(
            ..., jnp.maximum(
                jnp.dot(x_ref[...], w_ref[...],
                        preferred_element_type=jnp.float32) + b_ref[...], 0.0)),
        grid=(grid,),
        in_specs=[
            pl.BlockSpec((r, 16), lambda i: (i, 0)),
            pl.BlockSpec((16, 16), lambda i: (0, 0)),
            pl.BlockSpec((1, 16), lambda i: (0, 0)),
        ],
        out_specs=pl.BlockSpec((r, 16), lambda i: (i, 0)),
        out_shape=jax.ShapeDtypeStruct((n, 16), jnp.float32),
    )(x, w1t, b1)


def _tail(sums, cnts, h, wlt, bl, wrt, w2t, b2, n):
    r = 2048
    grid = sums.shape[1] // r

    def body(s_ref, c_ref, h_ref, wl_ref, bl_ref, wr_ref, w2_ref, b2_ref, o_ref):
        ssum = s_ref[0] + s_ref[1]
        cnt = c_ref[0] + c_ref[1]
        mean = ssum / jnp.maximum(cnt, 1.0)
        conv = (jnp.dot(mean, wl_ref[...], preferred_element_type=jnp.float32)
                + bl_ref[...]
                + jnp.dot(h_ref[...], wr_ref[...],
                          preferred_element_type=jnp.float32))
        h2 = jnp.maximum(conv, 0.0)
        o_ref[...] = (jnp.dot(h2, w2_ref[...], preferred_element_type=jnp.float32)
                      + b2_ref[...])

    return pl.pallas_call(
        body,
        grid=(grid,),
        in_specs=[
            pl.BlockSpec((2, r, 16), lambda i: (0, i, 0)),
            pl.BlockSpec((2, r, 1), lambda i: (0, i, 0)),
            pl.BlockSpec((r, 16), lambda i: (i, 0)),
            pl.BlockSpec((16, 32), lambda i: (0, 0)),
            pl.BlockSpec((1, 32), lambda i: (0, 0)),
            pl.BlockSpec((16, 32), lambda i: (0, 0)),
            pl.BlockSpec((32, 32), lambda i: (0, 0)),
            pl.BlockSpec((1, 32), lambda i: (0, 0)),
        ],
        out_specs=pl.BlockSpec((r, 32), lambda i: (i, 0)),
        out_shape=jax.ShapeDtypeStruct((n, 32), jnp.float32),
    )(sums, cnts, h, wlt, bl, wrt, w2t, b2)


def kernel(x, edge_index, W1, b1, Wl, bl, Wr, W2, b2):
    n = x.shape[0]
    e = edge_index.shape[1]

    sc_agg, n_acc = _build_sc_agg(n, e)

    h = _lin1(x, W1.T, b1.reshape(1, 16))

    src = edge_index[0].astype(jnp.int32)
    dst = edge_index[1].astype(jnp.int32)

    zrows = n_acc // _NTILES
    zsum = jnp.zeros((zrows, 16), jnp.float32)
    zcnt = jnp.zeros((zrows,), jnp.float32)
    ones = jnp.ones((800,), jnp.float32)

    sums, cnts = sc_agg(h, src, dst, zsum, zcnt, ones)
    cnts3 = cnts.reshape(_NCORES, n_acc, 1)

    return _tail(sums, cnts3, h, Wl.T, bl.reshape(1, 32), Wr.T, W2.T,
                 b2.reshape(1, 32), n)


# speedup vs baseline: 46.1065x; 1.2940x over previous
import functools

import jax
import jax.numpy as jnp
from jax import lax
from jax.experimental import pallas as pl
from jax.experimental.pallas import tpu as pltpu
from jax.experimental.pallas import tpu_sc as plsc

_NTILES = 16
_NCORES = 2
_NW = _NTILES * _NCORES


def _round_up(a, b):
    return (a + b - 1) // b * b


@functools.lru_cache(maxsize=None)
def _build_sc_agg(n_nodes, n_edges):
    n_acc = _round_up(n_nodes + 96, _NTILES * 128)
    zrows = n_acc // _NTILES

    ept = n_edges // _NW
    EB = 2000
    NBLK = ept // EB
    GCH = 400
    NCH = EB // GCH
    CCH = 1000
    NCC = EB // CCH
    G = 2

    mesh = plsc.VectorSubcoreMesh(core_axis_name="c", subcore_axis_name="s")

    @functools.partial(
        pl.kernel,
        out_type=[jax.ShapeDtypeStruct((_NCORES, n_acc, 16), jnp.float32)],
        mesh=mesh,
        compiler_params=pltpu.CompilerParams(use_tc_tiling_on_sc=False),
        scratch_types=[
            pltpu.VMEM((EB,), jnp.int32),
            pltpu.VMEM((EB,), jnp.int32),
            pltpu.VMEM((EB,), jnp.int32),
            pltpu.VMEM((G, GCH, 16), jnp.float32),
            pltpu.VMEM((CCH,), jnp.float32),
            pltpu.VMEM((128,), jnp.float32),
            pltpu.VMEM((128, 16), jnp.float32),
            pltpu.VMEM_SHARED((n_acc, 16), jnp.float32),
            pltpu.VMEM_SHARED((n_acc,), jnp.float32),
            pltpu.SemaphoreType.DMA((G,)),
            pltpu.SemaphoreType.DMA((G,)),
            pltpu.SemaphoreType.DMA,
        ],
    )
    def sc_agg(h_hbm, src_hbm, dst_hbm, zsum_hbm, zcnt_hbm, ones_hbm,
               out_mean,
               src_v, dst_v, dsto_v, rows, ones_v, cnt_v, mrow_v,
               sum_sh, cnt_sh, gsem, ssem, csem):
        c = lax.axis_index("c")
        s = lax.axis_index("s")

        pltpu.sync_copy(zsum_hbm, sum_sh.at[pl.ds(s * zrows, zrows)])
        pltpu.sync_copy(zcnt_hbm, cnt_sh.at[pl.ds(s * zrows, zrows)])
        pltpu.sync_copy(ones_hbm, ones_v)
        plsc.subcore_barrier()

        base = (c * _NTILES + s) * ept
        base_oth = ((1 - c) * _NTILES + s) * ept
        for blk in range(NBLK):
            pltpu.sync_copy(src_hbm.at[pl.ds(base + blk * EB, EB)], src_v)
            pltpu.sync_copy(dst_hbm.at[pl.ds(base + blk * EB, EB)], dst_v)
            pltpu.sync_copy(dst_hbm.at[pl.ds(base_oth + blk * EB, EB)], dsto_v)

            for t in range(NCC):
                pltpu.async_copy(
                    ones_v, cnt_sh.at[dst_v.at[pl.ds(t * CCH, CCH)]],
                    csem, add=True)
            for t in range(NCC):
                pltpu.async_copy(
                    ones_v, cnt_sh.at[dsto_v.at[pl.ds(t * CCH, CCH)]],
                    csem, add=True)

            for g in range(G):
                pltpu.async_copy(h_hbm.at[src_v.at[pl.ds(g * GCH, GCH)]],
                                 rows.at[g], gsem.at[g])

            def gstep(j, carry):
                slot = lax.rem(j, G)
                pltpu.make_async_copy(
                    h_hbm.at[src_v.at[pl.ds(j * GCH, GCH)]],
                    rows.at[slot], gsem.at[slot]).wait()
                pltpu.async_copy(rows.at[slot],
                                 sum_sh.at[dst_v.at[pl.ds(j * GCH, GCH)]],
                                 ssem.at[slot], add=True)

                @pl.when(j >= 1)
                def _recycle():
                    pslot = lax.rem(j - 1, G)
                    pltpu.make_async_copy(
                        rows.at[pslot],
                        sum_sh.at[dst_v.at[pl.ds((j - 1) * GCH, GCH)]],
                        ssem.at[pslot]).wait()

                    @pl.when(j + G - 1 < NCH)
                    def _fire():
                        pltpu.async_copy(
                            h_hbm.at[src_v.at[pl.ds((j + G - 1) * GCH, GCH)]],
                            rows.at[pslot], gsem.at[pslot])
                return carry

            lax.fori_loop(0, NCH, gstep, 0)
            pltpu.make_async_copy(
                rows.at[(NCH - 1) % G],
                sum_sh.at[dst_v.at[pl.ds((NCH - 1) * GCH, GCH)]],
                ssem.at[(NCH - 1) % G]).wait()

            for t in range(2 * NCC):
                pltpu.make_async_copy(
                    ones_v, cnt_sh.at[dst_v.at[pl.ds(0, CCH)]], csem).wait()

        plsc.subcore_barrier()

        def div_chunk(q, carry):
            row0 = s * zrows + q * 128
            pltpu.sync_copy(cnt_sh.at[pl.ds(row0, 128)], cnt_v)
            pltpu.sync_copy(sum_sh.at[pl.ds(row0, 128)], mrow_v)
            for g in range(8):
                c16 = cnt_v[pl.ds(g * 16, 16)]
                rv = 1.0 / jnp.maximum(c16, 1.0)
                for i in range(16):
                    idx = g * 16 + i
                    mrow_v[idx, :] = mrow_v[idx, :] * rv[i]
            pltpu.sync_copy(mrow_v, out_mean.at[c, pl.ds(row0, 128)])
            return carry

        lax.fori_loop(0, zrows // 128, div_chunk, 0)

    return sc_agg, n_acc


def _lin1_packed(xp, bd1, b1t):
    npk = xp.shape[0]
    r = 1024
    grid = pl.cdiv(npk, r)

    def body(x_ref, w_ref, b_ref, o_ref):
        o_ref[...] = jnp.maximum(
            jnp.dot(x_ref[...], w_ref[...],
                    preferred_element_type=jnp.float32) + b_ref[...], 0.0)

    return pl.pallas_call(
        body,
        grid=(grid,),
        in_specs=[
            pl.BlockSpec((r, 128), lambda i: (i, 0)),
            pl.BlockSpec((128, 128), lambda i: (0, 0)),
            pl.BlockSpec((1, 128), lambda i: (0, 0)),
        ],
        out_specs=pl.BlockSpec((r, 128), lambda i: (i, 0)),
        out_shape=jax.ShapeDtypeStruct((npk, 128), jnp.float32),
    )(xp, bd1, b1t)


def _tail_packed(mp, hp, bdl, blt, bdr, bd2, b2t):
    npk = hp.shape[0]
    r = 1024
    grid = pl.cdiv(npk, r)

    def body(m_ref, h_ref, wl_ref, bl_ref, wr_ref, w2_ref, b2_ref, o_ref):
        mean = m_ref[0] + m_ref[1]
        conv = (jnp.dot(mean, wl_ref[...], preferred_element_type=jnp.float32)
                + bl_ref[...]
                + jnp.dot(h_ref[...], wr_ref[...],
                          preferred_element_type=jnp.float32))
        h2 = jnp.maximum(conv, 0.0)
        o_ref[...] = (jnp.dot(h2, w2_ref[...], preferred_element_type=jnp.float32)
                      + b2_ref[...])

    return pl.pallas_call(
        body,
        grid=(grid,),
        in_specs=[
            pl.BlockSpec((2, r, 128), lambda i: (0, i, 0)),
            pl.BlockSpec((r, 128), lambda i: (i, 0)),
            pl.BlockSpec((128, 256), lambda i: (0, 0)),
            pl.BlockSpec((1, 256), lambda i: (0, 0)),
            pl.BlockSpec((128, 256), lambda i: (0, 0)),
            pl.BlockSpec((256, 256), lambda i: (0, 0)),
            pl.BlockSpec((1, 256), lambda i: (0, 0)),
        ],
        out_specs=pl.BlockSpec((r, 256), lambda i: (i, 0)),
        out_shape=jax.ShapeDtypeStruct((npk, 256), jnp.float32),
    )(mp, hp, bdl, blt, bdr, bd2, b2t)


def kernel(x, edge_index, W1, b1, Wl, bl, Wr, W2, b2):
    n = x.shape[0]
    e = edge_index.shape[1]
    npk = n // 8

    sc_agg, n_acc = _build_sc_agg(n, e)

    eye8 = jnp.eye(8, dtype=jnp.float32)
    xp = x.reshape(npk, 128)
    hp = _lin1_packed(xp, jnp.kron(eye8, W1.T), jnp.tile(b1, 8).reshape(1, 128))

    src = edge_index[0].astype(jnp.int32)
    dst = edge_index[1].astype(jnp.int32)

    zrows = n_acc // _NTILES
    zsum = jnp.zeros((zrows, 16), jnp.float32)
    zcnt = jnp.zeros((zrows,), jnp.float32)
    ones = jnp.ones((1000,), jnp.float32)

    (means,) = sc_agg(hp.reshape(n, 16), src, dst, zsum, zcnt, ones)
    mp = means.reshape(_NCORES, n_acc // 8, 128)

    outp = _tail_packed(
        mp, hp,
        jnp.kron(eye8, Wl.T), jnp.tile(bl, 8).reshape(1, 256),
        jnp.kron(eye8, Wr.T),
        jnp.kron(eye8, W2.T), jnp.tile(b2, 8).reshape(1, 256))
    return outp.reshape(n, 32)


# speedup vs baseline: 52.9293x; 1.1480x over previous
import functools

import jax
import jax.numpy as jnp
from jax import lax
from jax.experimental import pallas as pl
from jax.experimental.pallas import tpu as pltpu
from jax.experimental.pallas import tpu_sc as plsc

_NTILES = 16
_NCORES = 2
_NW = _NTILES * _NCORES


def _round_up(a, b):
    return (a + b - 1) // b * b


@functools.lru_cache(maxsize=None)
def _build_sc_agg(n_nodes, n_edges):
    n_acc = _round_up(n_nodes + 96, _NTILES * 128)
    zrows = n_acc // _NTILES

    ept = n_edges // _NW
    EB = 2000
    NBLK = ept // EB
    GCH = 400
    NCH = EB // GCH
    CCH = 1000
    NCC = EB // CCH
    G = 2

    mesh = plsc.VectorSubcoreMesh(core_axis_name="c", subcore_axis_name="s")

    @functools.partial(
        pl.kernel,
        out_type=[
            jax.ShapeDtypeStruct((_NCORES, n_acc, 16), jnp.float32),
            jax.ShapeDtypeStruct((_NCORES * n_acc,), jnp.float32),
        ],
        mesh=mesh,
        compiler_params=pltpu.CompilerParams(use_tc_tiling_on_sc=False),
        scratch_types=[
            pltpu.VMEM((EB,), jnp.int32),
            pltpu.VMEM((EB,), jnp.int32),
            pltpu.VMEM((EB,), jnp.int32),
            pltpu.VMEM((EB,), jnp.int32),
            pltpu.VMEM((G, GCH, 16), jnp.float32),
            pltpu.VMEM((CCH,), jnp.float32),
            pltpu.VMEM_SHARED((n_acc, 16), jnp.float32),
            pltpu.VMEM_SHARED((n_acc,), jnp.float32),
            pltpu.SemaphoreType.DMA((G,)),
            pltpu.SemaphoreType.DMA((G,)),
            pltpu.SemaphoreType.DMA,
            pltpu.SemaphoreType.DMA,
        ],
    )
    def sc_agg(h_hbm, src_hbm, dst_hbm, zsum_hbm, zcnt_hbm, ones_hbm,
               out_sum, out_cnt,
               src_a, dst_a, src_b, dst_b, rows, ones_v,
               sum_sh, cnt_sh, gsem, ssem, csem, isem):
        c = lax.axis_index("c")
        s = lax.axis_index("s")

        pltpu.sync_copy(zsum_hbm, sum_sh.at[pl.ds(s * zrows, zrows)])
        pltpu.sync_copy(zcnt_hbm, cnt_sh.at[pl.ds(s * zrows, zrows)])
        pltpu.sync_copy(ones_hbm, ones_v)
        plsc.subcore_barrier()

        base = (c * _NTILES + s) * ept
        pltpu.async_copy(src_hbm.at[pl.ds(base, EB)], src_a, isem)
        pltpu.async_copy(dst_hbm.at[pl.ds(base, EB)], dst_a, isem)
        for blk in range(NBLK):
            src_v, dst_v = (src_a, dst_a) if blk % 2 == 0 else (src_b, dst_b)
            nsrc, ndst = (src_b, dst_b) if blk % 2 == 0 else (src_a, dst_a)
            off = base + blk * EB
            pltpu.make_async_copy(src_hbm.at[pl.ds(off, EB)], src_v,
                                  isem).wait()
            pltpu.make_async_copy(dst_hbm.at[pl.ds(off, EB)], dst_v,
                                  isem).wait()
            if blk + 1 < NBLK:
                noff = base + (blk + 1) * EB
                pltpu.async_copy(src_hbm.at[pl.ds(noff, EB)], nsrc, isem)
                pltpu.async_copy(dst_hbm.at[pl.ds(noff, EB)], ndst, isem)

            for t in range(NCC):
                pltpu.async_copy(
                    ones_v, cnt_sh.at[dst_v.at[pl.ds(t * CCH, CCH)]],
                    csem, add=True)

            for g in range(G):
                pltpu.async_copy(h_hbm.at[src_v.at[pl.ds(g * GCH, GCH)]],
                                 rows.at[g], gsem.at[g])

            def gstep(j, carry):
                slot = lax.rem(j, G)
                pltpu.make_async_copy(
                    h_hbm.at[src_v.at[pl.ds(j * GCH, GCH)]],
                    rows.at[slot], gsem.at[slot]).wait()
                pltpu.async_copy(rows.at[slot],
                                 sum_sh.at[dst_v.at[pl.ds(j * GCH, GCH)]],
                                 ssem.at[slot], add=True)

                @pl.when(j >= 1)
                def _recycle():
                    pslot = lax.rem(j - 1, G)
                    pltpu.make_async_copy(
                        rows.at[pslot],
                        sum_sh.at[dst_v.at[pl.ds((j - 1) * GCH, GCH)]],
                        ssem.at[pslot]).wait()

                    @pl.when(j + G - 1 < NCH)
                    def _fire():
                        pltpu.async_copy(
                            h_hbm.at[src_v.at[pl.ds((j + G - 1) * GCH, GCH)]],
                            rows.at[pslot], gsem.at[pslot])
                return carry

            lax.fori_loop(0, NCH, gstep, 0)
            pltpu.make_async_copy(
                rows.at[(NCH - 1) % G],
                sum_sh.at[dst_v.at[pl.ds((NCH - 1) * GCH, GCH)]],
                ssem.at[(NCH - 1) % G]).wait()

            for t in range(NCC):
                pltpu.make_async_copy(
                    ones_v, cnt_sh.at[dst_v.at[pl.ds(0, CCH)]], csem).wait()

        plsc.subcore_barrier()
        pltpu.sync_copy(sum_sh.at[pl.ds(s * zrows, zrows)],
                        out_sum.at[c, pl.ds(s * zrows, zrows)])
        pltpu.sync_copy(cnt_sh.at[pl.ds(s * zrows, zrows)],
                        out_cnt.at[pl.ds(c * n_acc + s * zrows, zrows)])

    return sc_agg, n_acc


@functools.lru_cache(maxsize=None)
def _build_sc_mean(n_acc):
    rpt = n_acc // _NW
    CH = 448
    NQ = rpt // CH
    mesh = plsc.VectorSubcoreMesh(core_axis_name="c", subcore_axis_name="s")

    @functools.partial(
        pl.kernel,
        out_type=[jax.ShapeDtypeStruct((n_acc // 8, 128), jnp.float32)],
        mesh=mesh,
        compiler_params=pltpu.CompilerParams(use_tc_tiling_on_sc=False),
        scratch_types=[
            pltpu.VMEM((CH,), jnp.float32),
            pltpu.VMEM((CH,), jnp.float32),
            pltpu.VMEM((CH, 16), jnp.float32),
            pltpu.VMEM((CH, 16), jnp.float32),
            pltpu.VMEM((CH // 8, 128), jnp.float32),
        ],
    )
    def sc_mean(sum_hbm, cnt_hbm, out_m,
                c0_v, c1_v, s0_v, s1_v, ob_v):
        c = lax.axis_index("c")
        s = lax.axis_index("s")
        w = c * _NTILES + s
        for q in range(NQ):
            row0 = w * rpt + q * CH
            pltpu.sync_copy(cnt_hbm.at[pl.ds(row0, CH)], c0_v)
            pltpu.sync_copy(cnt_hbm.at[pl.ds(n_acc + row0, CH)], c1_v)
            pltpu.sync_copy(sum_hbm.at[0, pl.ds(row0, CH)], s0_v)
            pltpu.sync_copy(sum_hbm.at[1, pl.ds(row0, CH)], s1_v)

            def group(g, carry):
                c16 = c0_v[pl.ds(g * 16, 16)] + c1_v[pl.ds(g * 16, 16)]
                rv = 1.0 / jnp.maximum(c16, 1.0)
                for i in range(16):
                    idx = g * 16 + i
                    row = (s0_v[idx, :] + s1_v[idx, :]) * rv[i]
                    ob_v[g * 2 + i // 8, pl.ds((i % 8) * 16, 16)] = row
                return carry

            lax.fori_loop(0, CH // 16, group, 0)
            pltpu.sync_copy(ob_v, out_m.at[pl.ds(row0 // 8, CH // 8)])

    return sc_mean


def _lin1_packed(xp, bd1, b1t):
    npk = xp.shape[0]
    r = 1024
    grid = pl.cdiv(npk, r)

    def body(x_ref, w_ref, b_ref, o_ref):
        o_ref[...] = jnp.maximum(
            jnp.dot(x_ref[...], w_ref[...],
                    preferred_element_type=jnp.float32) + b_ref[...], 0.0)

    return pl.pallas_call(
        body,
        grid=(grid,),
        in_specs=[
            pl.BlockSpec((r, 128), lambda i: (i, 0)),
            pl.BlockSpec((128, 128), lambda i: (0, 0)),
            pl.BlockSpec((1, 128), lambda i: (0, 0)),
        ],
        out_specs=pl.BlockSpec((r, 128), lambda i: (i, 0)),
        out_shape=jax.ShapeDtypeStruct((npk, 128), jnp.float32),
    )(xp, bd1, b1t)


def _tail_packed(mp, hp, bdl, blt, bdr, bd2, b2t):
    npk = hp.shape[0]
    r = 1024
    grid = pl.cdiv(npk, r)

    def body(m_ref, h_ref, wl_ref, bl_ref, wr_ref, w2_ref, b2_ref, o_ref):
        mean = m_ref[...]
        conv = (jnp.dot(mean, wl_ref[...], preferred_element_type=jnp.float32)
                + bl_ref[...]
                + jnp.dot(h_ref[...], wr_ref[...],
                          preferred_element_type=jnp.float32))
        h2 = jnp.maximum(conv, 0.0)
        o_ref[...] = (jnp.dot(h2, w2_ref[...], preferred_element_type=jnp.float32)
                      + b2_ref[...])

    return pl.pallas_call(
        body,
        grid=(grid,),
        in_specs=[
            pl.BlockSpec((r, 128), lambda i: (i, 0)),
            pl.BlockSpec((r, 128), lambda i: (i, 0)),
            pl.BlockSpec((128, 256), lambda i: (0, 0)),
            pl.BlockSpec((1, 256), lambda i: (0, 0)),
            pl.BlockSpec((128, 256), lambda i: (0, 0)),
            pl.BlockSpec((256, 256), lambda i: (0, 0)),
            pl.BlockSpec((1, 256), lambda i: (0, 0)),
        ],
        out_specs=pl.BlockSpec((r, 256), lambda i: (i, 0)),
        out_shape=jax.ShapeDtypeStruct((npk, 256), jnp.float32),
    )(mp, hp, bdl, blt, bdr, bd2, b2t)


def kernel(x, edge_index, W1, b1, Wl, bl, Wr, W2, b2):
    n = x.shape[0]
    e = edge_index.shape[1]
    npk = n // 8

    sc_agg, n_acc = _build_sc_agg(n, e)

    eye8 = jnp.eye(8, dtype=jnp.float32)
    xp = x.reshape(npk, 128)
    hp = _lin1_packed(xp, jnp.kron(eye8, W1.T), jnp.tile(b1, 8).reshape(1, 128))

    src = edge_index[0].astype(jnp.int32)
    dst = edge_index[1].astype(jnp.int32)

    zrows = n_acc // _NTILES
    zsum = jnp.zeros((zrows, 16), jnp.float32)
    zcnt = jnp.zeros((zrows,), jnp.float32)
    ones = jnp.ones((1000,), jnp.float32)

    sums, cnts = sc_agg(hp.reshape(n, 16), src, dst, zsum, zcnt, ones)
    sc_mean = _build_sc_mean(n_acc)
    (mp,) = sc_mean(sums, cnts)

    outp = _tail_packed(
        mp, hp,
        jnp.kron(eye8, Wl.T), jnp.tile(bl, 8).reshape(1, 256),
        jnp.kron(eye8, Wr.T),
        jnp.kron(eye8, W2.T), jnp.tile(b2, 8).reshape(1, 256))
    return outp.reshape(n, 32)
